# baseline (speedup 1.0000x reference)
import jax
import jax.numpy as jnp
from jax import lax
from jax.experimental import pallas as pl
from jax.experimental.pallas import tpu as pltpu
from jax.experimental.pallas import tpu_sc as plsc

_B = 16384
_D = 64
_V = 100000
_NC = 2
_NS = 16
_NW = _NC * _NS
_FPW = _D // _NW
_SUB = 4096
_NSUB = _B // _SUB
_LANES = 16


def _film_body(ht_hbm, idx_hbm, gt_hbm, bt_hbm, outt_hbm,
               idx_v, tab_v, h_v, sem_tab, sem_h, sem_out, sem_idx):
    wid = lax.axis_index("s") * _NC + lax.axis_index("c")
    c0 = wid * _FPW

    def start_tab(tab_hbm, c):
        pltpu.async_copy(tab_hbm.at[c], tab_v, sem_tab)

    def start_idx(s):
        pltpu.async_copy(idx_hbm.at[pl.ds(s * _SUB, _SUB)],
                         idx_v.at[s % 2], sem_idx.at[s % 2])

    def wait_idx(s):
        pltpu.make_async_copy(idx_hbm.at[pl.ds(s * _SUB, _SUB)],
                              idx_v.at[s % 2], sem_idx.at[s % 2]).wait()

    start_tab(gt_hbm, c0)
    pltpu.async_copy(ht_hbm.at[c0], h_v, sem_h)
    start_idx(0)

    for f in range(_FPW):
        c = c0 + f
        for tab_hbm, is_mul in ((gt_hbm, True), (bt_hbm, False)):
            pltpu.make_async_copy(tab_hbm.at[c], tab_v, sem_tab).wait()
            if is_mul:
                pltpu.make_async_copy(ht_hbm.at[c], h_v, sem_h).wait()
            for s in range(_NSUB):
                wait_idx(s)
                last_scan = (f == _FPW - 1) and (not is_mul) and s == _NSUB - 1
                if not last_scan:
                    start_idx((s + 1) % _NSUB)
                slot = s % 2
                base = s * _SUB

                @plsc.parallel_loop(0, _SUB // _LANES, 1, unroll=8)
                def _(k):
                    iv = idx_v[slot, pl.ds(k * _LANES, _LANES)]
                    tv = plsc.load_gather(tab_v, [iv])
                    sl = pl.ds(base + k * _LANES, _LANES)
                    if is_mul:
                        h_v[sl] = h_v[sl] * tv
                    else:
                        h_v[sl] = h_v[sl] + tv

            if is_mul:
                start_tab(bt_hbm, c)
            else:
                pltpu.async_copy(h_v, outt_hbm.at[c], sem_out)
                if f + 1 < _FPW:
                    start_tab(gt_hbm, c + 1)
                    pltpu.make_async_copy(h_v, outt_hbm.at[c], sem_out).wait()
                    pltpu.async_copy(ht_hbm.at[c + 1], h_v, sem_h)

    pltpu.make_async_copy(h_v, outt_hbm.at[c0 + _FPW - 1], sem_out).wait()


@jax.jit
def _film(ht, idx, gt, bt):
    fn = pl.kernel(
        _film_body,
        mesh=plsc.VectorSubcoreMesh(core_axis_name="c", subcore_axis_name="s"),
        out_type=jax.ShapeDtypeStruct((_D, _B), jnp.float32),
        scratch_types=[
            pltpu.VMEM((2, _SUB), jnp.int32),
            pltpu.VMEM((_V,), jnp.float32),
            pltpu.VMEM((_B,), jnp.float32),
            pltpu.SemaphoreType.DMA,
            pltpu.SemaphoreType.DMA,
            pltpu.SemaphoreType.DMA,
            pltpu.SemaphoreType.DMA((2,)),
        ],
        compiler_params=pltpu.CompilerParams(
            needs_layout_passes=False,
            disable_bounds_checks=True,
        ),
    )
    return fn(ht, idx, gt, bt)


def kernel(h, idx, gamma, beta):
    outt = _film(h.T, idx.astype(jnp.int32), gamma.T, beta.T)
    return outt.T


# speedup vs baseline: 1.0773x; 1.0773x over previous
import jax
import jax.numpy as jnp
from jax import lax
from jax.experimental import pallas as pl
from jax.experimental.pallas import tpu as pltpu
from jax.experimental.pallas import tpu_sc as plsc

_B = 16384
_D = 64
_V = 100000
_NC = 2
_NS = 16
_NW = _NC * _NS
_FPW = _D // _NW
_SUB = 4096
_NSUB = _B // _SUB
_LANES = 16


def _film_body(ht_hbm, idx_hbm, gt_hbm, bt_hbm, outt_hbm,
               idx_v, tab_v, h_v, idx_sh, sem_tab, sem_h, sem_out, sem_idx):
    sid = lax.axis_index("s")
    wid = sid * _NC + lax.axis_index("c")
    c0 = wid * _FPW

    def start_tab(tab_hbm, c):
        pltpu.async_copy(tab_hbm.at[c], tab_v, sem_tab)

    def start_idx(s):
        pltpu.async_copy(idx_sh.at[pl.ds(s * _SUB, _SUB)],
                         idx_v.at[s % 2], sem_idx.at[s % 2])

    def wait_idx(s):
        pltpu.make_async_copy(idx_sh.at[pl.ds(s * _SUB, _SUB)],
                              idx_v.at[s % 2], sem_idx.at[s % 2]).wait()

    start_tab(gt_hbm, c0)
    pltpu.async_copy(ht_hbm.at[c0], h_v, sem_h)

    @pl.when(sid == 0)
    def _():
        pltpu.sync_copy(idx_hbm, idx_sh)

    plsc.subcore_barrier()
    start_idx(0)

    for f in range(_FPW):
        c = c0 + f
        for tab_hbm, is_mul in ((gt_hbm, True), (bt_hbm, False)):
            pltpu.make_async_copy(tab_hbm.at[c], tab_v, sem_tab).wait()
            if is_mul:
                pltpu.make_async_copy(ht_hbm.at[c], h_v, sem_h).wait()
            for s in range(_NSUB):
                wait_idx(s)
                last_scan = (f == _FPW - 1) and (not is_mul) and s == _NSUB - 1
                if not last_scan:
                    start_idx((s + 1) % _NSUB)
                slot = s % 2
                base = s * _SUB

                @plsc.parallel_loop(0, _SUB // _LANES, 1, unroll=8)
                def _(k):
                    iv = idx_v[slot, pl.ds(k * _LANES, _LANES)]
                    tv = plsc.load_gather(tab_v, [iv])
                    sl = pl.ds(base + k * _LANES, _LANES)
                    if is_mul:
                        h_v[sl] = h_v[sl] * tv
                    else:
                        h_v[sl] = h_v[sl] + tv

            if is_mul:
                start_tab(bt_hbm, c)
            else:
                pltpu.async_copy(h_v, outt_hbm.at[c], sem_out)
                if f + 1 < _FPW:
                    start_tab(gt_hbm, c + 1)
                    pltpu.make_async_copy(h_v, outt_hbm.at[c], sem_out).wait()
                    pltpu.async_copy(ht_hbm.at[c + 1], h_v, sem_h)

    pltpu.make_async_copy(h_v, outt_hbm.at[c0 + _FPW - 1], sem_out).wait()


@jax.jit
def _film(ht, idx, gt, bt):
    fn = pl.kernel(
        _film_body,
        mesh=plsc.VectorSubcoreMesh(core_axis_name="c", subcore_axis_name="s"),
        out_type=jax.ShapeDtypeStruct((_D, _B), jnp.float32),
        scratch_types=[
            pltpu.VMEM((2, _SUB), jnp.int32),
            pltpu.VMEM((_V,), jnp.float32),
            pltpu.VMEM((_B,), jnp.float32),
            pltpu.VMEM_SHARED((_B,), jnp.int32),
            pltpu.SemaphoreType.DMA,
            pltpu.SemaphoreType.DMA,
            pltpu.SemaphoreType.DMA,
            pltpu.SemaphoreType.DMA((2,)),
        ],
        compiler_params=pltpu.CompilerParams(
            needs_layout_passes=False,
            disable_bounds_checks=True,
        ),
    )
    return fn(ht, idx, gt, bt)


def kernel(h, idx, gamma, beta):
    outt = _film(h.T, idx.astype(jnp.int32), gamma.T, beta.T)
    return outt.T


# speedup vs baseline: 1.0801x; 1.0026x over previous
import jax
import jax.numpy as jnp
from jax import lax
from jax.experimental import pallas as pl
from jax.experimental.pallas import tpu as pltpu
from jax.experimental.pallas import tpu_sc as plsc

_B = 16384
_D = 64
_V = 100000
_NC = 2
_NS = 16
_NW = _NC * _NS
_FPW = _D // _NW
_SUB = 4096
_NSUB = _B // _SUB
_LANES = 16


def _film_body(ht_hbm, idx_hbm, gt_hbm, bt_hbm, outt_hbm,
               idx_v, tab_v, h_v, idx_sh, sem_tab, sem_h, sem_out, sem_idx):
    sid = lax.axis_index("s")
    wid = sid * _NC + lax.axis_index("c")
    c0 = wid * _FPW

    def start_tab(tab_hbm, c):
        pltpu.async_copy(tab_hbm.at[c], tab_v, sem_tab)

    def start_idx(s):
        pltpu.async_copy(idx_sh.at[pl.ds(s * _SUB, _SUB)],
                         idx_v.at[s % 2], sem_idx.at[s % 2])

    def wait_idx(s):
        pltpu.make_async_copy(idx_sh.at[pl.ds(s * _SUB, _SUB)],
                              idx_v.at[s % 2], sem_idx.at[s % 2]).wait()

    start_tab(gt_hbm, c0)
    pltpu.async_copy(ht_hbm.at[c0], h_v, sem_h)

    @pl.when(sid == 0)
    def _():
        pltpu.sync_copy(idx_hbm, idx_sh)

    plsc.subcore_barrier()
    start_idx(0)

    for f in range(_FPW):
        c = c0 + f
        for tab_hbm, is_mul in ((gt_hbm, True), (bt_hbm, False)):
            pltpu.make_async_copy(tab_hbm.at[c], tab_v, sem_tab).wait()
            if is_mul:
                pltpu.make_async_copy(ht_hbm.at[c], h_v, sem_h).wait()
            for s in range(_NSUB):
                wait_idx(s)
                last_scan = (f == _FPW - 1) and (not is_mul) and s == _NSUB - 1
                if not last_scan:
                    start_idx((s + 1) % _NSUB)
                slot = s % 2
                base = s * _SUB

                @plsc.parallel_loop(0, _SUB // _LANES, 1, unroll=8)
                def _(k):
                    iv = idx_v[slot, pl.ds(k * _LANES, _LANES)]
                    tv = plsc.load_gather(tab_v, [iv])
                    sl = pl.ds(base + k * _LANES, _LANES)
                    if is_mul:
                        h_v[sl] = h_v[sl] * tv
                    else:
                        h_v[sl] = h_v[sl] + tv

            if is_mul:
                start_tab(bt_hbm, c)
            else:
                pltpu.async_copy(h_v, outt_hbm.at[c], sem_out)
                if f + 1 < _FPW:
                    start_tab(gt_hbm, c + 1)
                    pltpu.make_async_copy(h_v, outt_hbm.at[c], sem_out).wait()
                    pltpu.async_copy(ht_hbm.at[c + 1], h_v, sem_h)

    pltpu.make_async_copy(h_v, outt_hbm.at[c0 + _FPW - 1], sem_out).wait()


@jax.jit
def _film(ht, idx, gt, bt):
    fn = pl.kernel(
        _film_body,
        mesh=plsc.VectorSubcoreMesh(core_axis_name="c", subcore_axis_name="s"),
        out_type=jax.ShapeDtypeStruct((_D, _B), jnp.float32),
        scratch_types=[
            pltpu.VMEM((2, _SUB), jnp.int32),
            pltpu.VMEM((_V,), jnp.float32),
            pltpu.VMEM((_B,), jnp.float32),
            pltpu.VMEM_SHARED((_B,), jnp.int32),
            pltpu.SemaphoreType.DMA,
            pltpu.SemaphoreType.DMA,
            pltpu.SemaphoreType.DMA,
            pltpu.SemaphoreType.DMA((2,)),
        ],
        compiler_params=pltpu.CompilerParams(
            needs_layout_passes=False,
            disable_bounds_checks=True,
            skip_device_barrier=True,
        ),
    )
    return fn(ht, idx, gt, bt)


def kernel(h, idx, gamma, beta):
    outt = _film(h.T, idx.astype(jnp.int32), gamma.T, beta.T)
    return outt.T
